# baseline (speedup 1.0000x reference)
import functools
import jax
import jax.numpy as jnp
from jax import lax
from jax.experimental import pallas as pl
from jax.experimental.pallas import tpu as pltpu
from jax.experimental.pallas import tpu_sc as plsc

_ROWS = 4096
_COLS = 16384
_NC = 2
_NS = 16
_NW = _NC * _NS
_RPW = _ROWS // _NW
_CHUNK = 2
_NCHUNK = _RPW // _CHUNK
_NPAIR = _NCHUNK // 2


def kernel(x):
    mesh = plsc.VectorSubcoreMesh(core_axis_name="c", subcore_axis_name="s")

    @functools.partial(
        pl.kernel,
        out_type=jax.ShapeDtypeStruct((_ROWS, _COLS), jnp.float32),
        mesh=mesh,
        scratch_types=[
            pltpu.VMEM((2, _CHUNK, _COLS), jnp.float32),
            pltpu.SemaphoreType.DMA((2,)),
            pltpu.SemaphoreType.DMA((2,)),
        ],
    )
    def sc_copy(x_hbm, out_hbm, buf, load_sems, store_sems):
        wid = lax.axis_index("s") * _NC + lax.axis_index("c")
        base = wid * _RPW

        def load(c, slot):
            return pltpu.make_async_copy(
                x_hbm.at[pl.ds(base + c * _CHUNK, _CHUNK), :],
                buf.at[slot],
                load_sems.at[slot],
            )

        def store(c, slot):
            return pltpu.make_async_copy(
                buf.at[slot],
                out_hbm.at[pl.ds(base + c * _CHUNK, _CHUNK), :],
                store_sems.at[slot],
            )

        load(0, 0).start()
        load(1, 1).start()

        @pl.loop(0, _NPAIR - 1)
        def _steady(j):
            a = 2 * j
            b = a + 1
            load(a, 0).wait()
            store(a, 0).start()
            load(b, 1).wait()
            store(b, 1).start()
            store(a, 0).wait()
            load(a + 2, 0).start()
            store(b, 1).wait()
            load(b + 2, 1).start()

        a = _NCHUNK - 2
        b = _NCHUNK - 1
        load(a, 0).wait()
        store(a, 0).start()
        load(b, 1).wait()
        store(b, 1).start()
        store(a, 0).wait()
        store(b, 1).wait()

    return sc_copy(x)


# speedup vs baseline: 1.0160x; 1.0160x over previous
import functools
import jax
import jax.numpy as jnp
from jax import lax
from jax.experimental import pallas as pl
from jax.experimental.pallas import tpu as pltpu
from jax.experimental.pallas import tpu_sc as plsc

_ROWS = 4096
_COLS = 16384
_NC = 2
_NS = 16
_NW = _NC * _NS
_RPW = _ROWS // _NW
_SLOTS = 4
_NITER = _RPW // _SLOTS


def kernel(x):
    mesh = plsc.VectorSubcoreMesh(core_axis_name="c", subcore_axis_name="s")

    @functools.partial(
        pl.kernel,
        out_type=jax.ShapeDtypeStruct((_ROWS, _COLS), jnp.float32),
        mesh=mesh,
        scratch_types=[
            pltpu.VMEM((_SLOTS, 1, _COLS), jnp.float32),
            pltpu.SemaphoreType.DMA((_SLOTS,)),
            pltpu.SemaphoreType.DMA((_SLOTS,)),
        ],
    )
    def sc_copy(x_hbm, out_hbm, buf, load_sems, store_sems):
        wid = lax.axis_index("s") * _NC + lax.axis_index("c")
        base = wid * _RPW

        def load(row, slot):
            return pltpu.make_async_copy(
                x_hbm.at[pl.ds(base + row, 1), :],
                buf.at[slot],
                load_sems.at[slot],
            )

        def store(row, slot):
            return pltpu.make_async_copy(
                buf.at[slot],
                out_hbm.at[pl.ds(base + row, 1), :],
                store_sems.at[slot],
            )

        for s in range(_SLOTS):
            load(s, s).start()

        @pl.loop(0, _NITER - 1)
        def _steady(j):
            r0 = j * _SLOTS
            for s in range(_SLOTS):
                load(r0 + s, s).wait()
                store(r0 + s, s).start()
            for s in range(_SLOTS):
                store(r0 + s, s).wait()
                load(r0 + s + _SLOTS, s).start()

        r0 = (_NITER - 1) * _SLOTS
        for s in range(_SLOTS):
            load(r0 + s, s).wait()
            store(r0 + s, s).start()
        for s in range(_SLOTS):
            store(r0 + s, s).wait()

    return sc_copy(x)
